# speedup vs baseline: 7.3862x; 7.3862x over previous
import functools
import math

import jax
import jax.numpy as jnp
from jax import lax
from jax.experimental import pallas as pl
from jax.experimental.pallas import tpu as pltpu
from jax.experimental.pallas import tpu_sc as plsc

NUM_CORES = 2
NUM_SUBCORES = 16
NW = NUM_CORES * NUM_SUBCORES
LANES = 16
CHUNK = 128


def _build_gather(nchunks: int, d: int, n_pad: int):
    scale = math.sqrt(d)
    mesh = plsc.VectorSubcoreMesh(core_axis_name="c", subcore_axis_name="s")

    @functools.partial(
        pl.kernel,
        out_type=jax.ShapeDtypeStruct((n_pad, d), jnp.float32),
        mesh=mesh,
        scratch_types=[
            pltpu.VMEM((nchunks, CHUNK), jnp.int32),
            pltpu.VMEM((CHUNK, d), jnp.float32),
            pltpu.VMEM((CHUNK, d), jnp.float32),
            pltpu.SemaphoreType.DMA,
            pltpu.SemaphoreType.DMA,
        ],
    )
    def gather_kernel(idx_hbm, table_hbm, out_hbm, idx_v, buf0, buf1,
                      sem0, sem1):
        wid = lax.axis_index("s") * NUM_CORES + lax.axis_index("c")
        row_base = wid * (nchunks * CHUNK)

        pltpu.sync_copy(idx_hbm.at[wid], idx_v)

        pltpu.async_copy(table_hbm.at[idx_v.at[0]], buf0, sem0)
        pltpu.async_copy(table_hbm.at[idx_v.at[1]], buf1, sem1)

        def do_chunk(cur, buf, sem):
            pltpu.make_async_copy(table_hbm.at[idx_v.at[cur]], buf, sem).wait()

            def scale_row(i, _):
                for j in range(d // LANES):
                    sl = pl.ds(j * LANES, LANES)
                    buf[i, sl] = buf[i, sl] * scale
                return _

            lax.fori_loop(0, CHUNK, scale_row, None)

            pltpu.sync_copy(buf, out_hbm.at[pl.ds(row_base + cur * CHUNK,
                                                  CHUNK)])

            nxt = cur + 2
            @pl.when(nxt < nchunks)
            def _():
                pltpu.async_copy(table_hbm.at[idx_v.at[nxt]], buf, sem)

        def body(k, carry):
            cur = k * 2
            do_chunk(cur, buf0, sem0)
            do_chunk(cur + 1, buf1, sem1)
            return carry

        lax.fori_loop(0, nchunks // 2, body, None)

    return gather_kernel


def kernel(position_ids, table):
    b, s = position_ids.shape
    v, d = table.shape
    n = b * s

    per_worker = -(-n // NW)
    nchunks = -(-per_worker // CHUNK)
    if nchunks % 2:
        nchunks += 1
    n_pad = NW * nchunks * CHUNK

    idx = position_ids.reshape(n).astype(jnp.int32)
    if n_pad != n:
        idx = jnp.pad(idx, (0, n_pad - n))
    idx3 = idx.reshape(NW, nchunks, CHUNK)

    out = _build_gather(nchunks, d, n_pad)(idx3, table)
    if n_pad != n:
        out = out[:n]
    return out.reshape(b, s, d)


# speedup vs baseline: 7.8131x; 1.0578x over previous
import functools
import math

import jax
import jax.numpy as jnp
from jax import lax
from jax.experimental import pallas as pl
from jax.experimental.pallas import tpu as pltpu
from jax.experimental.pallas import tpu_sc as plsc

NUM_CORES = 2
NUM_SUBCORES = 16
NW = NUM_CORES * NUM_SUBCORES
LANES = 16
CHUNK = 128


def _build_gather(nchunks: int, d: int, n_pad: int):
    scale = math.sqrt(d)
    mesh = plsc.VectorSubcoreMesh(core_axis_name="c", subcore_axis_name="s")

    @functools.partial(
        pl.kernel,
        out_type=jax.ShapeDtypeStruct((n_pad, d), jnp.float32),
        mesh=mesh,
        scratch_types=[
            pltpu.VMEM((nchunks, CHUNK), jnp.int32),
            pltpu.VMEM((CHUNK, d), jnp.float32),
            pltpu.VMEM((CHUNK, d), jnp.float32),
            pltpu.VMEM((CHUNK, d), jnp.float32),
            pltpu.VMEM((CHUNK, d), jnp.float32),
            pltpu.SemaphoreType.DMA,
            pltpu.SemaphoreType.DMA,
            pltpu.SemaphoreType.DMA,
            pltpu.SemaphoreType.DMA,
        ],
    )
    def gather_kernel(idx_hbm, table_hbm, out_hbm, idx_v, g0, g1, w0, w1,
                      gs0, gs1, ws0, ws1):
        wid = lax.axis_index("s") * NUM_CORES + lax.axis_index("c")
        row_base = wid * (nchunks * CHUNK)

        def out_slice(c):
            return out_hbm.at[pl.ds(row_base + c * CHUNK, CHUNK)]

        pltpu.sync_copy(idx_hbm.at[wid], idx_v)

        pltpu.async_copy(table_hbm.at[idx_v.at[0]], g0, gs0)
        pltpu.async_copy(table_hbm.at[idx_v.at[1]], g1, gs1)

        def do_chunk(cur, gbuf, gsem, wbuf, wsem):
            pltpu.make_async_copy(table_hbm.at[idx_v.at[cur]], gbuf,
                                  gsem).wait()
            @pl.when(cur >= 2)
            def _():
                pltpu.make_async_copy(wbuf, out_slice(cur - 2), wsem).wait()

            @plsc.parallel_loop(0, CHUNK, unroll=4)
            def _(i):
                for j in range(d // LANES):
                    sl = pl.ds(j * LANES, LANES)
                    wbuf[i, sl] = gbuf[i, sl] * scale

            pltpu.async_copy(wbuf, out_slice(cur), wsem)
            nxt = cur + 2
            @pl.when(nxt < nchunks)
            def _():
                pltpu.async_copy(table_hbm.at[idx_v.at[nxt]], gbuf, gsem)

        def body(k, carry):
            cur = k * 2
            do_chunk(cur, g0, gs0, w0, ws0)
            do_chunk(cur + 1, g1, gs1, w1, ws1)
            return carry

        lax.fori_loop(0, nchunks // 2, body, None)

        pltpu.make_async_copy(w0, out_slice(nchunks - 2), ws0).wait()
        pltpu.make_async_copy(w1, out_slice(nchunks - 1), ws1).wait()

    return gather_kernel


def kernel(position_ids, table):
    b, s = position_ids.shape
    v, d = table.shape
    n = b * s

    per_worker = -(-n // NW)
    nchunks = -(-per_worker // CHUNK)
    if nchunks % 2:
        nchunks += 1
    n_pad = NW * nchunks * CHUNK

    idx = position_ids.reshape(n).astype(jnp.int32)
    if n_pad != n:
        idx = jnp.pad(idx, (0, n_pad - n))
    idx3 = idx.reshape(NW, nchunks, CHUNK)

    out = _build_gather(nchunks, d, n_pad)(idx3, table)
    if n_pad != n:
        out = out[:n]
    return out.reshape(b, s, d)
